# baseline (speedup 1.0000x reference)
import jax
import jax.numpy as jnp
from jax.experimental import pallas as pl

_N_CLASSES = 1000
_ROWS_PER_BLOCK = 512


def _onehot_block(x_ref, o_ref):
    idx = x_ref[0, 0, :]
    iota = jax.lax.broadcasted_iota(jnp.int32, (_ROWS_PER_BLOCK, _N_CLASSES), 1)
    o_ref[...] = (idx[:, None] == iota).astype(jnp.float32)


def kernel(x):
    b, s = x.shape
    total = b * s
    nb = total // _ROWS_PER_BLOCK
    x_flat = x.reshape(nb, 1, _ROWS_PER_BLOCK)
    out = pl.pallas_call(
        _onehot_block,
        grid=(nb,),
        in_specs=[pl.BlockSpec((1, 1, _ROWS_PER_BLOCK), lambda i: (i, 0, 0))],
        out_specs=pl.BlockSpec((_ROWS_PER_BLOCK, _N_CLASSES), lambda i: (i, 0)),
        out_shape=jax.ShapeDtypeStruct((total, _N_CLASSES), jnp.float32),
    )(x_flat)
    return out.reshape(b, s, _N_CLASSES)


# speedup vs baseline: 1.5732x; 1.5732x over previous
import jax
import jax.numpy as jnp
from jax.experimental import pallas as pl

_N_CLASSES = 1000
_B_BLOCK = 32


def _onehot_block(x_ref, o_ref):
    idx = x_ref[...]
    iota = jax.lax.broadcasted_iota(
        jnp.int32, (_B_BLOCK, idx.shape[1], _N_CLASSES), 2
    )
    o_ref[...] = (idx[:, :, None] == iota).astype(jnp.float32)


def kernel(x):
    b, s = x.shape
    nb = b // _B_BLOCK
    return pl.pallas_call(
        _onehot_block,
        grid=(nb,),
        in_specs=[pl.BlockSpec((_B_BLOCK, s), lambda i: (i, 0))],
        out_specs=pl.BlockSpec((_B_BLOCK, s, _N_CLASSES), lambda i: (i, 0, 0)),
        out_shape=jax.ShapeDtypeStruct((b, s, _N_CLASSES), jnp.float32),
    )(x)


# speedup vs baseline: 1.6105x; 1.0237x over previous
import jax
import jax.numpy as jnp
from jax.experimental import pallas as pl

_N_CLASSES = 1000
_B_BLOCK = 128


def _onehot_block(x_ref, o_ref):
    idx = x_ref[...]
    iota = jax.lax.broadcasted_iota(
        jnp.int32, (_B_BLOCK, idx.shape[1], _N_CLASSES), 2
    )
    o_ref[...] = (idx[:, :, None] == iota).astype(jnp.float32)


def kernel(x):
    b, s = x.shape
    nb = b // _B_BLOCK
    return pl.pallas_call(
        _onehot_block,
        grid=(nb,),
        in_specs=[pl.BlockSpec((_B_BLOCK, s), lambda i: (i, 0))],
        out_specs=pl.BlockSpec((_B_BLOCK, s, _N_CLASSES), lambda i: (i, 0, 0)),
        out_shape=jax.ShapeDtypeStruct((b, s, _N_CLASSES), jnp.float32),
    )(x)


# speedup vs baseline: 1.6141x; 1.0022x over previous
import jax
import jax.numpy as jnp
from jax.experimental import pallas as pl
from jax.experimental.pallas import tpu as pltpu

_N_CLASSES = 1000
_B_BLOCK = 32
_NBUF = 8


def _onehot_body(x_ref, o_ref, vmem, sem):
    i = pl.program_id(0)
    nb = pl.num_programs(0)
    slot = jax.lax.rem(i, _NBUF)

    def copy(j, s):
        return pltpu.make_async_copy(
            vmem.at[s], o_ref.at[pl.ds(j * _B_BLOCK, _B_BLOCK)], sem.at[s]
        )

    @pl.when(i >= _NBUF)
    def _wait_prev():
        copy(i - _NBUF, slot).wait()

    idx = x_ref[...]
    iota = jax.lax.broadcasted_iota(
        jnp.int32, (_B_BLOCK, idx.shape[1], _N_CLASSES), 2
    )
    vmem[slot] = (idx[:, :, None] == iota).astype(jnp.float32)
    copy(i, slot).start()

    @pl.when(i == nb - 1)
    def _drain():
        for k in range(_NBUF):
            j = i - (_NBUF - 1) + k
            copy(j, jax.lax.rem(j, _NBUF)).wait()


def kernel(x):
    b, s = x.shape
    nb = b // _B_BLOCK
    return pl.pallas_call(
        _onehot_body,
        grid=(nb,),
        in_specs=[pl.BlockSpec((_B_BLOCK, s), lambda i: (i, 0))],
        out_specs=pl.BlockSpec(memory_space=pl.MemorySpace.ANY),
        out_shape=jax.ShapeDtypeStruct((b, s, _N_CLASSES), jnp.float32),
        scratch_shapes=[
            pltpu.VMEM((_NBUF, _B_BLOCK, s, _N_CLASSES), jnp.float32),
            pltpu.SemaphoreType.DMA((_NBUF,)),
        ],
    )(x)
